# Initial kernel scaffold; baseline (speedup 1.0000x reference)
import jax
import jax.numpy as jnp
from jax.experimental import pallas as pl


def kernel(input_predicted, input_encoded):
    raise NotImplementedError("write your pallas kernel here")


# speedup vs baseline: 2.4845x; 2.4845x over previous
import functools

import jax
import jax.numpy as jnp
from jax import lax
from jax.experimental import pallas as pl
from jax.experimental.pallas import tpu as pltpu
from jax.experimental.pallas import tpu_sc as plsc

BS = 64
SEQ = 128
D = 512
SP = 12
NF = 128
NI = SEQ - SP
R = BS * NI
C = BS * SEQ
TEMP = 0.07

NW = 32
RPW = R // NW
EPW = RPW * (1 + NF)
CHUNK = 128
NCH = (EPW + CHUNK - 1) // CHUNK
EPW_PAD = NCH * CHUNK
FIRES = 13
NGROUP = NCH // FIRES


@functools.lru_cache(maxsize=1)
def _sample_indices():
    rng = jax.random.key(42)
    seqs, bats = [], []
    for i in range(NI):
        rng, ka, kb = jax.random.split(rng, 3)
        s = jax.random.randint(ka, (BS * NF,), 0, SEQ - 1)
        s2 = jnp.where(s < i + SP, s, s + 1)
        b = jax.random.randint(kb, (BS * NF,), 0, BS)
        seqs.append(s2)
        bats.append(b)
    s2a = jnp.stack(seqs)
    bba = jnp.stack(bats)
    col_neg = (bba * SEQ + s2a).reshape(NI, BS, NF).transpose(1, 0, 2)
    col_pos = (jnp.arange(BS)[:, None] * SEQ
               + (jnp.arange(NI)[None, :] + SP))[..., None]
    cols = jnp.concatenate([col_pos, col_neg], axis=2).reshape(R, 1 + NF)
    flat = cols.astype(jnp.int32) + (jnp.arange(R, dtype=jnp.int32) * C)[:, None]
    flat = flat.reshape(NW, EPW)
    flat = jnp.pad(flat, ((0, 0), (0, EPW_PAD - EPW)))
    return flat.reshape(NW, NCH, CHUNK)


RBLK = 256
CBLK = 1024


def _sim_body(p_ref, e_ref, s_ref):
    a = p_ref[...]
    b = e_ref[...]
    an = a / jnp.maximum(jnp.sqrt(jnp.sum(a * a, axis=1, keepdims=True)), 1e-12)
    bn = b / jnp.maximum(jnp.sqrt(jnp.sum(b * b, axis=1, keepdims=True)), 1e-12)
    s_ref[...] = lax.dot_general(
        an, bn, (((1,), (1,)), ((), ())),
        preferred_element_type=jnp.float32,
        precision=lax.Precision.HIGHEST)


def _similarity(p, e):
    return pl.pallas_call(
        _sim_body,
        grid=(C // CBLK, R // RBLK),
        in_specs=[
            pl.BlockSpec((RBLK, D), lambda j, r: (r, 0)),
            pl.BlockSpec((CBLK, D), lambda j, r: (j, 0)),
        ],
        out_specs=pl.BlockSpec((RBLK, CBLK), lambda j, r: (r, j)),
        out_shape=jax.ShapeDtypeStruct((R, C), jnp.float32),
        compiler_params=pltpu.CompilerParams(
            dimension_semantics=("arbitrary", "arbitrary")),
    )(p, e)


def _gather_body(s_hbm, idx_hbm, out_hbm, idx_v, g_v, sem):
    nc = 2
    wid = lax.axis_index("s") * nc + lax.axis_index("c")
    pltpu.sync_copy(idx_hbm.at[wid], idx_v)

    def group(g, carry):
        for t in range(FIRES):
            j = g * FIRES + t
            dst = g_v.at[pl.ds(pl.multiple_of(j * CHUNK, CHUNK), CHUNK)]
            pltpu.async_copy(s_hbm.at[idx_v.at[j]], dst, sem)
        return carry

    lax.fori_loop(0, NGROUP, group, 0)
    pltpu.make_async_copy(out_hbm.at[pl.ds(0, EPW_PAD)], g_v, sem).wait()
    base = pl.multiple_of(wid * EPW_PAD, EPW_PAD)
    pltpu.sync_copy(g_v, out_hbm.at[pl.ds(base, EPW_PAD)])


def _gather(s_flat, idx):
    k = pl.kernel(
        _gather_body,
        mesh=plsc.VectorSubcoreMesh(core_axis_name="c", subcore_axis_name="s"),
        out_type=jax.ShapeDtypeStruct((NW * EPW_PAD,), jnp.float32),
        scratch_types=[
            pltpu.VMEM((NCH, CHUNK), jnp.int32),
            pltpu.VMEM((EPW_PAD,), jnp.float32),
            pltpu.SemaphoreType.DMA,
        ],
    )
    return k(s_flat, idx)


LBLK = 128


def _loss_body(g_ref, loss_ref, acc_ref):
    a = pl.program_id(0)
    x = g_ref[...] * (1.0 / TEMP)
    xp = x[:, 0:1]
    mn = jnp.max(x[:, 1:], axis=1, keepdims=True)
    m = jnp.maximum(mn, xp)
    lse = m + jnp.log(jnp.sum(jnp.exp(x - m), axis=1, keepdims=True))
    loss_part = jnp.sum(lse - xp) * (1.0 / BS)
    tp_part = jnp.sum((xp >= mn).astype(jnp.float32))

    @pl.when(a == 0)
    def _():
        loss_ref[0, 0] = 0.0
        acc_ref[0, 0] = 0.0

    loss_ref[0, 0] += loss_part
    acc_ref[0, 0] += tp_part

    @pl.when(a == pl.num_programs(0) - 1)
    def _():
        acc_ref[0, 0] = acc_ref[0, 0] * (1.0 / (BS * NI))


def _loss(g2):
    return pl.pallas_call(
        _loss_body,
        grid=(R // LBLK,),
        in_specs=[pl.BlockSpec((LBLK, 1 + NF), lambda a: (a, 0))],
        out_specs=[
            pl.BlockSpec(memory_space=pltpu.SMEM),
            pl.BlockSpec(memory_space=pltpu.SMEM),
        ],
        out_shape=[
            jax.ShapeDtypeStruct((1, 1), jnp.float32),
            jax.ShapeDtypeStruct((1, 1), jnp.float32),
        ],
        compiler_params=pltpu.CompilerParams(
            dimension_semantics=("arbitrary",)),
    )(g2)


def kernel(input_predicted, input_encoded):
    p = input_predicted[:, :NI, :].reshape(R, D)
    e = input_encoded.reshape(C, D)
    s = _similarity(p, e)
    g = _gather(s.reshape(-1), _sample_indices())
    g2 = g.reshape(NW, EPW_PAD)[:, :EPW].reshape(R, 1 + NF)
    loss, acc = _loss(g2)
    return loss[0, 0], acc[0, 0]


# speedup vs baseline: 11.4095x; 4.5923x over previous
import numpy as np

import jax
import jax.numpy as jnp
from jax import lax
from jax.experimental import pallas as pl
from jax.experimental.pallas import tpu as pltpu
from jax.experimental.pallas import tpu_sc as plsc

BS = 64
SEQ = 128
D = 512
SP = 12
NF = 128
NI = SEQ - SP
R = BS * NI
C = BS * SEQ
TEMP = 0.07

NW = 32
RPW = R // NW
EPW = RPW * (1 + NF)
CHUNK = 128
NCH = (EPW + CHUNK - 1) // CHUNK
EPW_PAD = NCH * CHUNK
FIRES = 13
NGROUP = NCH // FIRES


def _sample_indices_np():
    rng = jax.random.key(42)
    seqs, bats = [], []
    for i in range(NI):
        rng, ka, kb = jax.random.split(rng, 3)
        s = jax.random.randint(ka, (BS * NF,), 0, SEQ - 1)
        s2 = jnp.where(s < i + SP, s, s + 1)
        b = jax.random.randint(kb, (BS * NF,), 0, BS)
        seqs.append(np.asarray(s2))
        bats.append(np.asarray(b))
    s2a = np.stack(seqs)
    bba = np.stack(bats)
    col_neg = (bba * SEQ + s2a).reshape(NI, BS, NF).transpose(1, 0, 2)
    col_pos = (np.arange(BS)[:, None] * SEQ
               + (np.arange(NI)[None, :] + SP))[..., None]
    cols = np.concatenate([col_pos, col_neg], axis=2).reshape(R, 1 + NF)
    r = np.arange(R)[:, None]
    addr = (r * C + cols).astype(np.int32)
    flat = addr.reshape(NW, EPW)
    flat = np.pad(flat, ((0, 0), (0, EPW_PAD - EPW)))
    return flat.reshape(NW, NCH, CHUNK)


try:
    _IDX_DEV = jax.devices("cpu")[0]
except RuntimeError:
    _IDX_DEV = None

if _IDX_DEV is not None:
    with jax.default_device(_IDX_DEV):
        _IDX_NP = _sample_indices_np()
else:
    _IDX_NP = _sample_indices_np()


def _sample_indices():
    return jnp.asarray(_IDX_NP)


RBLK = 256
CBLK = 1024


def _sim_body(p_ref, e_ref, s_ref):
    a = p_ref[...]
    b = e_ref[...]
    an = a / jnp.maximum(jnp.sqrt(jnp.sum(a * a, axis=1, keepdims=True)), 1e-12)
    bn = b / jnp.maximum(jnp.sqrt(jnp.sum(b * b, axis=1, keepdims=True)), 1e-12)
    s_ref[...] = lax.dot_general(
        an, bn, (((1,), (1,)), ((), ())),
        preferred_element_type=jnp.float32,
        precision=lax.Precision.HIGHEST)


def _similarity(p, e):
    return pl.pallas_call(
        _sim_body,
        grid=(C // CBLK, R // RBLK),
        in_specs=[
            pl.BlockSpec((RBLK, D), lambda j, r: (r, 0)),
            pl.BlockSpec((CBLK, D), lambda j, r: (j, 0)),
        ],
        out_specs=pl.BlockSpec((RBLK, CBLK), lambda j, r: (r, j)),
        out_shape=jax.ShapeDtypeStruct((R, C), jnp.float32),
        compiler_params=pltpu.CompilerParams(
            dimension_semantics=("arbitrary", "arbitrary")),
    )(p, e)


def _gather_body(s_hbm, idx_hbm, out_hbm, idx_v, g_v, sem):
    nc = 2
    wid = lax.axis_index("s") * nc + lax.axis_index("c")
    pltpu.sync_copy(idx_hbm.at[wid], idx_v)

    def group(g, carry):
        for t in range(FIRES):
            j = g * FIRES + t
            dst = g_v.at[pl.ds(pl.multiple_of(j * CHUNK, CHUNK), CHUNK)]
            pltpu.async_copy(s_hbm.at[idx_v.at[j]], dst, sem)
        return carry

    lax.fori_loop(0, NGROUP, group, 0)
    pltpu.make_async_copy(out_hbm.at[pl.ds(0, EPW_PAD)], g_v, sem).wait()
    base = pl.multiple_of(wid * EPW_PAD, EPW_PAD)
    pltpu.sync_copy(g_v, out_hbm.at[pl.ds(base, EPW_PAD)])


def _gather(s_flat, idx):
    k = pl.kernel(
        _gather_body,
        mesh=plsc.VectorSubcoreMesh(core_axis_name="c", subcore_axis_name="s"),
        out_type=jax.ShapeDtypeStruct((NW * EPW_PAD,), jnp.float32),
        scratch_types=[
            pltpu.VMEM((NCH, CHUNK), jnp.int32),
            pltpu.VMEM((EPW_PAD,), jnp.float32),
            pltpu.SemaphoreType.DMA,
        ],
    )
    return k(s_flat, idx)


LBLK = 128


def _loss_body(g_ref, loss_ref, acc_ref):
    a = pl.program_id(0)
    x = g_ref[...] * (1.0 / TEMP)
    xp = x[:, 0:1]
    mn = jnp.max(x[:, 1:], axis=1, keepdims=True)
    m = jnp.maximum(mn, xp)
    lse = m + jnp.log(jnp.sum(jnp.exp(x - m), axis=1, keepdims=True))
    loss_part = jnp.sum(lse - xp) * (1.0 / BS)
    tp_part = jnp.sum((xp >= mn).astype(jnp.float32))

    @pl.when(a == 0)
    def _():
        loss_ref[0, 0] = 0.0
        acc_ref[0, 0] = 0.0

    loss_ref[0, 0] += loss_part
    acc_ref[0, 0] += tp_part

    @pl.when(a == pl.num_programs(0) - 1)
    def _():
        acc_ref[0, 0] = acc_ref[0, 0] * (1.0 / (BS * NI))


def _loss(g2):
    return pl.pallas_call(
        _loss_body,
        grid=(R // LBLK,),
        in_specs=[pl.BlockSpec((LBLK, 1 + NF), lambda a: (a, 0))],
        out_specs=[
            pl.BlockSpec(memory_space=pltpu.SMEM),
            pl.BlockSpec(memory_space=pltpu.SMEM),
        ],
        out_shape=[
            jax.ShapeDtypeStruct((1, 1), jnp.float32),
            jax.ShapeDtypeStruct((1, 1), jnp.float32),
        ],
        compiler_params=pltpu.CompilerParams(
            dimension_semantics=("arbitrary",)),
    )(g2)


def kernel(input_predicted, input_encoded):
    p = input_predicted[:, :NI, :].reshape(R, D)
    e = input_encoded.reshape(C, D)
    s = _similarity(p, e)
    g = _gather(s.reshape(-1), _sample_indices())
    g2 = g.reshape(NW, EPW_PAD)[:, :EPW].reshape(R, 1 + NF)
    loss, acc = _loss(g2)
    return loss[0, 0], acc[0, 0]


# speedup vs baseline: 14.5958x; 1.2793x over previous
import numpy as np

import jax
import jax.numpy as jnp
from jax import lax
from jax.experimental import pallas as pl
from jax.experimental.pallas import tpu as pltpu
from jax.experimental.pallas import tpu_sc as plsc

BS = 64
SEQ = 128
D = 512
SP = 12
NF = 128
NI = SEQ - SP
R = BS * NI
C = BS * SEQ
TEMP = 0.07

NW = 32
RPW = R // NW
EPW = RPW * (1 + NF)
CHUNK = 128
NCH = (EPW + CHUNK - 1) // CHUNK
EPW_PAD = NCH * CHUNK
FIRES = 13
NGROUP = NCH // FIRES


def _sample_indices_np():
    rng = jax.random.key(42)
    seqs, bats = [], []
    for i in range(NI):
        rng, ka, kb = jax.random.split(rng, 3)
        s = jax.random.randint(ka, (BS * NF,), 0, SEQ - 1)
        s2 = jnp.where(s < i + SP, s, s + 1)
        b = jax.random.randint(kb, (BS * NF,), 0, BS)
        seqs.append(np.asarray(s2))
        bats.append(np.asarray(b))
    s2a = np.stack(seqs)
    bba = np.stack(bats)
    col_neg = (bba * SEQ + s2a).reshape(NI, BS, NF).transpose(1, 0, 2)
    col_pos = (np.arange(BS)[:, None] * SEQ
               + (np.arange(NI)[None, :] + SP))[..., None]
    cols = np.concatenate([col_pos, col_neg], axis=2).reshape(R, 1 + NF)
    r = np.arange(R)[:, None]
    addr = (r * C + cols).astype(np.int32)
    flat = addr.reshape(NW, EPW)
    flat = np.pad(flat, ((0, 0), (0, EPW_PAD - EPW)))
    return flat.reshape(NW, NCH, CHUNK)


try:
    _IDX_DEV = jax.devices("cpu")[0]
except RuntimeError:
    _IDX_DEV = None

if _IDX_DEV is not None:
    with jax.default_device(_IDX_DEV):
        _IDX_NP = _sample_indices_np()
else:
    _IDX_NP = _sample_indices_np()


def _sample_indices():
    return jnp.asarray(_IDX_NP)


RBLK = 256
CBLK = 1024


def _sim_body(p_ref, e_ref, s_ref):
    a = p_ref[...]
    b = e_ref[...]
    an = a / jnp.maximum(jnp.sqrt(jnp.sum(a * a, axis=1, keepdims=True)), 1e-12)
    bn = b / jnp.maximum(jnp.sqrt(jnp.sum(b * b, axis=1, keepdims=True)), 1e-12)
    ah = an.astype(jnp.bfloat16)
    al = (an - ah.astype(jnp.float32)).astype(jnp.bfloat16)
    bh = bn.astype(jnp.bfloat16)
    bl = (bn - bh.astype(jnp.float32)).astype(jnp.bfloat16)

    def dot(x, y):
        return lax.dot_general(
            x, y, (((1,), (1,)), ((), ())),
            preferred_element_type=jnp.float32)

    s_ref[...] = dot(ah, bh) + (dot(ah, bl) + dot(al, bh))


def _similarity(p, e):
    return pl.pallas_call(
        _sim_body,
        grid=(C // CBLK, R // RBLK),
        in_specs=[
            pl.BlockSpec((RBLK, D), lambda j, r: (r, 0)),
            pl.BlockSpec((CBLK, D), lambda j, r: (j, 0)),
        ],
        out_specs=pl.BlockSpec((RBLK, CBLK), lambda j, r: (r, j)),
        out_shape=jax.ShapeDtypeStruct((R, C), jnp.float32),
        compiler_params=pltpu.CompilerParams(
            dimension_semantics=("arbitrary", "arbitrary")),
    )(p, e)


def _gather_body(s_hbm, idx_hbm, out_hbm, idx_v, g_v, sem):
    nc = 2
    wid = lax.axis_index("s") * nc + lax.axis_index("c")
    pltpu.sync_copy(idx_hbm.at[wid], idx_v)

    def group(g, carry):
        for t in range(FIRES):
            j = g * FIRES + t
            dst = g_v.at[pl.ds(pl.multiple_of(j * CHUNK, CHUNK), CHUNK)]
            pltpu.async_copy(s_hbm.at[idx_v.at[j]], dst, sem)
        return carry

    lax.fori_loop(0, NGROUP, group, 0)
    pltpu.make_async_copy(out_hbm.at[pl.ds(0, EPW_PAD)], g_v, sem).wait()
    base = pl.multiple_of(wid * EPW_PAD, EPW_PAD)
    pltpu.sync_copy(g_v, out_hbm.at[pl.ds(base, EPW_PAD)])


def _gather(s_flat, idx):
    k = pl.kernel(
        _gather_body,
        mesh=plsc.VectorSubcoreMesh(core_axis_name="c", subcore_axis_name="s"),
        out_type=jax.ShapeDtypeStruct((NW * EPW_PAD,), jnp.float32),
        scratch_types=[
            pltpu.VMEM((NCH, CHUNK), jnp.int32),
            pltpu.VMEM((EPW_PAD,), jnp.float32),
            pltpu.SemaphoreType.DMA,
        ],
    )
    return k(s_flat, idx)


LBLK = 128


def _loss_body(g_ref, loss_ref, acc_ref):
    a = pl.program_id(0)
    x = g_ref[...] * (1.0 / TEMP)
    xp = x[:, 0:1]
    mn = jnp.max(x[:, 1:], axis=1, keepdims=True)
    m = jnp.maximum(mn, xp)
    lse = m + jnp.log(jnp.sum(jnp.exp(x - m), axis=1, keepdims=True))
    loss_part = jnp.sum(lse - xp) * (1.0 / BS)
    tp_part = jnp.sum((xp >= mn).astype(jnp.float32))

    @pl.when(a == 0)
    def _():
        loss_ref[0, 0] = 0.0
        acc_ref[0, 0] = 0.0

    loss_ref[0, 0] += loss_part
    acc_ref[0, 0] += tp_part

    @pl.when(a == pl.num_programs(0) - 1)
    def _():
        acc_ref[0, 0] = acc_ref[0, 0] * (1.0 / (BS * NI))


def _loss(g2):
    return pl.pallas_call(
        _loss_body,
        grid=(R // LBLK,),
        in_specs=[pl.BlockSpec((LBLK, 1 + NF), lambda a: (a, 0))],
        out_specs=[
            pl.BlockSpec(memory_space=pltpu.SMEM),
            pl.BlockSpec(memory_space=pltpu.SMEM),
        ],
        out_shape=[
            jax.ShapeDtypeStruct((1, 1), jnp.float32),
            jax.ShapeDtypeStruct((1, 1), jnp.float32),
        ],
        compiler_params=pltpu.CompilerParams(
            dimension_semantics=("arbitrary",)),
    )(g2)


def kernel(input_predicted, input_encoded):
    p = input_predicted[:, :NI, :].reshape(R, D)
    e = input_encoded.reshape(C, D)
    s = _similarity(p, e)
    g = _gather(s.reshape(-1), _sample_indices())
    g2 = g.reshape(NW, EPW_PAD)[:, :EPW].reshape(R, 1 + NF)
    loss, acc = _loss(g2)
    return loss[0, 0], acc[0, 0]


# speedup vs baseline: 19.1180x; 1.3098x over previous
import numpy as np

import jax
import jax.numpy as jnp
from jax import lax
from jax.experimental import pallas as pl
from jax.experimental.pallas import tpu as pltpu
from jax.experimental.pallas import tpu_sc as plsc

BS = 64
SEQ = 128
D = 512
SP = 12
NF = 128
NI = SEQ - SP
R = BS * NI
C = BS * SEQ
TEMP = 0.07

NW = 32
RPW = R // NW
EPW = RPW * (1 + NF)
CHUNK = 128
NCH = (EPW + CHUNK - 1) // CHUNK
EPW_PAD = NCH * CHUNK
FIRES = 13
NGROUP = NCH // FIRES


def _sample_indices_np():
    rng = jax.random.key(42)
    seqs, bats = [], []
    for i in range(NI):
        rng, ka, kb = jax.random.split(rng, 3)
        s = jax.random.randint(ka, (BS * NF,), 0, SEQ - 1)
        s2 = jnp.where(s < i + SP, s, s + 1)
        b = jax.random.randint(kb, (BS * NF,), 0, BS)
        seqs.append(np.asarray(s2))
        bats.append(np.asarray(b))
    s2a = np.stack(seqs)
    bba = np.stack(bats)
    col_neg = (bba * SEQ + s2a).reshape(NI, BS, NF).transpose(1, 0, 2)
    col_pos = (np.arange(BS)[:, None] * SEQ
               + (np.arange(NI)[None, :] + SP))[..., None]
    cols = np.concatenate([col_pos, col_neg], axis=2).reshape(R, 1 + NF)
    r = np.arange(R)[:, None]
    addr = ((r // 8) * 65536 + (cols // 128) * 1024
            + (r % 8) * 128 + (cols % 128)).astype(np.int32)
    flat = addr.reshape(NW, EPW)
    flat = np.pad(flat, ((0, 0), (0, EPW_PAD - EPW)))
    return flat.reshape(NW, NCH, CHUNK)


try:
    _IDX_DEV = jax.devices("cpu")[0]
except RuntimeError:
    _IDX_DEV = None

if _IDX_DEV is not None:
    with jax.default_device(_IDX_DEV):
        _IDX_NP = _sample_indices_np()
else:
    _IDX_NP = _sample_indices_np()


def _sample_indices():
    return jnp.asarray(_IDX_NP)


RBLK = 256
CBLK = 1024


def _sim_body(p_ref, e_ref, s_ref):
    a = p_ref[...]
    b = e_ref[...]
    an = a / jnp.maximum(jnp.sqrt(jnp.sum(a * a, axis=1, keepdims=True)), 1e-12)
    bn = b / jnp.maximum(jnp.sqrt(jnp.sum(b * b, axis=1, keepdims=True)), 1e-12)
    ah = an.astype(jnp.bfloat16)
    al = (an - ah.astype(jnp.float32)).astype(jnp.bfloat16)
    bh = bn.astype(jnp.bfloat16)
    bl = (bn - bh.astype(jnp.float32)).astype(jnp.bfloat16)

    def dot(x, y):
        return lax.dot_general(
            x, y, (((1,), (1,)), ((), ())),
            preferred_element_type=jnp.float32)

    res = dot(ah, bh) + (dot(ah, bl) + dot(al, bh))
    r4 = res.reshape(RBLK // 8, 8, CBLK // 128, 128)
    s_ref[...] = r4.transpose(0, 2, 1, 3).reshape(RBLK // 8, CBLK // 128 * 8, 128)


def _similarity(p, e):
    return pl.pallas_call(
        _sim_body,
        grid=(C // CBLK, R // RBLK),
        in_specs=[
            pl.BlockSpec((RBLK, D), lambda j, r: (r, 0)),
            pl.BlockSpec((CBLK, D), lambda j, r: (j, 0)),
        ],
        out_specs=pl.BlockSpec((RBLK // 8, CBLK // 128 * 8, 128),
                               lambda j, r: (r, j, 0)),
        out_shape=jax.ShapeDtypeStruct((R // 8, C // 128 * 8, 128), jnp.float32),
        compiler_params=pltpu.CompilerParams(
            dimension_semantics=("arbitrary", "arbitrary")),
    )(p, e)


def _gather_body(s_hbm, idx_hbm, out_hbm, idx_v, g_v, sem):
    nc = 2
    wid = lax.axis_index("s") * nc + lax.axis_index("c")
    pltpu.sync_copy(idx_hbm.at[wid], idx_v)

    def group(g, carry):
        for t in range(FIRES):
            j = g * FIRES + t
            dst = g_v.at[pl.ds(pl.multiple_of(j * CHUNK, CHUNK), CHUNK)]
            pltpu.async_copy(s_hbm.at[idx_v.at[j]], dst, sem)
        return carry

    lax.fori_loop(0, NGROUP, group, 0)
    pltpu.make_async_copy(out_hbm.at[pl.ds(0, EPW_PAD)], g_v, sem).wait()
    base = pl.multiple_of(wid * EPW_PAD, EPW_PAD)
    pltpu.sync_copy(g_v, out_hbm.at[pl.ds(base, EPW_PAD)])


def _gather(s_flat, idx):
    k = pl.kernel(
        _gather_body,
        mesh=plsc.VectorSubcoreMesh(core_axis_name="c", subcore_axis_name="s"),
        out_type=jax.ShapeDtypeStruct((NW * EPW_PAD,), jnp.float32),
        scratch_types=[
            pltpu.VMEM((NCH, CHUNK), jnp.int32),
            pltpu.VMEM((EPW_PAD,), jnp.float32),
            pltpu.SemaphoreType.DMA,
        ],
    )
    return k(s_flat, idx)


LBLK = 128


def _loss_body(g_ref, loss_ref, acc_ref):
    a = pl.program_id(0)
    x = g_ref[...] * (1.0 / TEMP)
    xp = x[:, 0:1]
    mn = jnp.max(x[:, 1:], axis=1, keepdims=True)
    m = jnp.maximum(mn, xp)
    lse = m + jnp.log(jnp.sum(jnp.exp(x - m), axis=1, keepdims=True))
    loss_part = jnp.sum(lse - xp) * (1.0 / BS)
    tp_part = jnp.sum((xp >= mn).astype(jnp.float32))

    @pl.when(a == 0)
    def _():
        loss_ref[0, 0] = 0.0
        acc_ref[0, 0] = 0.0

    loss_ref[0, 0] += loss_part
    acc_ref[0, 0] += tp_part

    @pl.when(a == pl.num_programs(0) - 1)
    def _():
        acc_ref[0, 0] = acc_ref[0, 0] * (1.0 / (BS * NI))


def _loss(g2):
    return pl.pallas_call(
        _loss_body,
        grid=(R // LBLK,),
        in_specs=[pl.BlockSpec((LBLK, 1 + NF), lambda a: (a, 0))],
        out_specs=[
            pl.BlockSpec(memory_space=pltpu.SMEM),
            pl.BlockSpec(memory_space=pltpu.SMEM),
        ],
        out_shape=[
            jax.ShapeDtypeStruct((1, 1), jnp.float32),
            jax.ShapeDtypeStruct((1, 1), jnp.float32),
        ],
        compiler_params=pltpu.CompilerParams(
            dimension_semantics=("arbitrary",)),
    )(g2)


def kernel(input_predicted, input_encoded):
    p = input_predicted[:, :NI, :].reshape(R, D)
    e = input_encoded.reshape(C, D)
    s = _similarity(p, e)
    g = _gather(s.reshape(-1), _sample_indices())
    g2 = g.reshape(NW, EPW_PAD)[:, :EPW].reshape(R, 1 + NF)
    loss, acc = _loss(g2)
    return loss[0, 0], acc[0, 0]
